# baseline (speedup 1.0000x reference)
import functools

import jax
import jax.numpy as jnp
import numpy as np
from jax import lax
from jax.experimental import pallas as pl
from jax.experimental.pallas import tpu as pltpu
from jax.experimental.pallas import tpu_sc as plsc

N_NODES = 10000
N_EDGES = 320000
D = 128
P_DROP = 0.5

NC = 2
NS = 16
NT = NC * NS
G = 128

SINK = N_NODES
NPAD = 10240
RPT = NPAD // NS
NPM = 10112
RPM = NPM // NS
ROWB = 1024
NBLK = NPAD // ROWB


def _rotl32(x, d):
    return ((x << np.uint32(d)) | (x >> np.uint32(32 - d))).astype(np.uint32)


def _threefry2x32(k0, k1, x0, x1):
    ks = [np.uint32(k0), np.uint32(k1),
          np.uint32(np.uint32(k0) ^ np.uint32(k1) ^ np.uint32(0x1BD11BDA))]
    rotations = [(13, 15, 26, 6), (17, 29, 16, 24)]
    x0 = (x0 + ks[0]).astype(np.uint32)
    x1 = (x1 + ks[1]).astype(np.uint32)
    for i in range(5):
        for r in rotations[i % 2]:
            x0 = (x0 + x1).astype(np.uint32)
            x1 = _rotl32(x1, r)
            x1 = (x1 ^ x0).astype(np.uint32)
        x0 = (x0 + ks[(i + 1) % 3]).astype(np.uint32)
        x1 = (x1 + ks[(i + 2) % 3] + np.uint32(i + 1)).astype(np.uint32)
    return x0, x1


def _drop_edge_mask(seed, n, p_keep):
    b1, b2 = _threefry2x32(np.uint32(0), np.uint32(seed),
                           np.zeros(n, dtype=np.uint32),
                           np.arange(n, dtype=np.uint32))
    bits = (b1 ^ b2).astype(np.uint32)
    u = ((bits >> np.uint32(9)) | np.uint32(0x3F800000)).view(np.float32)
    return (u - np.float32(1.0)) < np.float32(p_keep)


_MASK = _drop_edge_mask(42, N_EDGES, 1.0 - P_DROP)
_POS = np.nonzero(_MASK)[0].astype(np.int32)
K = _POS.shape[0]
NCH = -(-K // (NT * G))
K_PAD = NT * G * NCH
_POS_PAD = np.concatenate(
    [_POS, np.full(K_PAD - K, N_EDGES, dtype=np.int32)])

NCH0 = 78
NCH1 = 2 * NCH - NCH0

NCHD = -(-N_EDGES // (NT * G))
E_PAD = NT * G * NCHD


_sc_mesh = plsc.VectorSubcoreMesh(
    core_axis_name="c", subcore_axis_name="s", num_cores=NC, num_subcores=NS)


@functools.partial(
    pl.kernel,
    out_type=jax.ShapeDtypeStruct((NC * 2 * NPAD,), jnp.float32),
    mesh=_sc_mesh,
    scratch_types=[
        pltpu.VMEM((NCHD, G), jnp.int32),
        pltpu.VMEM((NCHD, G), jnp.int32),
        pltpu.VMEM((G,), jnp.float32),
        pltpu.VMEM((RPT,), jnp.float32),
        pltpu.VMEM_SHARED((NPAD,), jnp.float32),
        pltpu.VMEM_SHARED((NPAD,), jnp.float32),
    ],
)
def _deg_kernel(srcd, dstd, out, sidx, didx, ones1, zb, hs, hd):
    c = lax.axis_index("c")
    s = lax.axis_index("s")
    w = c * NS + s

    pltpu.sync_copy(srcd.at[w], sidx)
    pltpu.sync_copy(dstd.at[w], didx)

    def _fill_ones(i, _):
        ones1[pl.ds(i * 16, 16)] = jnp.ones((16,), jnp.float32)
        return 0

    def _fill_zeros(i, _):
        zb[pl.ds(i * 16, 16)] = jnp.zeros((16,), jnp.float32)
        return 0

    lax.fori_loop(0, G // 16, _fill_ones, 0)
    lax.fori_loop(0, RPT // 16, _fill_zeros, 0)

    base = s * RPT
    pltpu.sync_copy(zb, hs.at[pl.ds(base, RPT)])
    pltpu.sync_copy(zb, hd.at[pl.ds(base, RPT)])
    plsc.subcore_barrier()

    def _chunk(j, _):
        pltpu.sync_copy(ones1, hs.at[sidx.at[j]], add=True)
        pltpu.sync_copy(ones1, hd.at[didx.at[j]], add=True)
        return 0

    lax.fori_loop(0, NCHD, _chunk, 0)
    plsc.subcore_barrier()

    pltpu.sync_copy(hs.at[pl.ds(base, RPT)],
                    out.at[pl.ds(c * 2 * NPAD + base, RPT)])
    pltpu.sync_copy(hd.at[pl.ds(base, RPT)],
                    out.at[pl.ds(c * 2 * NPAD + NPAD + base, RPT)])


@functools.partial(
    pl.kernel,
    out_type=jax.ShapeDtypeStruct((NC, NPM, D), jnp.float32),
    mesh=_sc_mesh,
    scratch_types=[
        pltpu.VMEM((NCH0, G), jnp.int32),
        pltpu.VMEM((2, G), jnp.int32),
        pltpu.VMEM((2, G), jnp.int32),
        pltpu.VMEM((G, D), jnp.float32),
        pltpu.VMEM((G, D), jnp.float32),
        pltpu.VMEM_SHARED((NPM, D), jnp.float32),
        pltpu.SemaphoreType.DMA,
        pltpu.SemaphoreType.DMA,
    ],
)
def _mp_kernel(h, kept0, kept1, out,
               pval, sbuf, dbuf, rows0, rows1, agg, gsem, isem):
    c = lax.axis_index("c")
    s = lax.axis_index("s")
    ncx = jnp.where(c == 0, NCH0, NCH1)

    @pl.when(c == 0)
    def _():
        pltpu.async_copy(kept0.at[s], pval.at[pl.ds(0, NCH0)], isem)

    @pl.when(c == 1)
    def _():
        pltpu.async_copy(kept1.at[s], pval.at[pl.ds(0, NCH1)], isem)

    def _zero_row(i, _):
        for l in range(D // 16):
            rows0[i, pl.ds(l * 16, 16)] = jnp.zeros((16,), jnp.float32)
        return 0

    lax.fori_loop(0, G, _zero_row, 0)

    base = s * RPM
    for k in range(RPM // G):
        pltpu.sync_copy(rows0, agg.at[pl.ds(base + k * G, G)])
    rem = RPM % G
    if rem:
        off = base + (RPM // G) * G
        pltpu.sync_copy(rows0.at[pl.ds(0, rem)], agg.at[pl.ds(off, rem)])

    @pl.when(c == 0)
    def _():
        pltpu.make_async_copy(kept0.at[s], pval.at[pl.ds(0, NCH0)], isem).wait()

    @pl.when(c == 1)
    def _():
        pltpu.make_async_copy(kept1.at[s], pval.at[pl.ds(0, NCH1)], isem).wait()

    plsc.subcore_barrier()

    def _decode(j, b):
        for l in range(G // 16):
            v = pval[j, pl.ds(l * 16, 16)]
            sbuf[b, pl.ds(l * 16, 16)] = v & jnp.int32(0x3FFF)
            dbuf[b, pl.ds(l * 16, 16)] = lax.shift_right_logical(
                v, jnp.int32(14))

    _decode(0, 0)
    pltpu.async_copy(h.at[sbuf.at[0]], rows0, gsem)
    _decode(1, 1)
    npair = ncx // 2

    def _pair(jj, _):
        j0 = 2 * jj
        j1 = j0 + 1
        pltpu.async_copy(h.at[sbuf.at[1]], rows1, gsem)
        pltpu.make_async_copy(h.at[sbuf.at[0]], rows0, gsem).wait()
        pltpu.sync_copy(rows0, agg.at[dbuf.at[0]], add=True)

        @pl.when(jj < npair - 1)
        def _():
            _decode(j0 + 2, 0)
            pltpu.async_copy(h.at[sbuf.at[0]], rows0, gsem)

        pltpu.make_async_copy(h.at[sbuf.at[1]], rows1, gsem).wait()
        pltpu.sync_copy(rows1, agg.at[dbuf.at[1]], add=True)

        @pl.when(jj < npair - 1)
        def _():
            _decode(j1 + 2, 1)
        return 0

    lax.fori_loop(0, npair, _pair, 0)
    plsc.subcore_barrier()

    pltpu.sync_copy(agg.at[pl.ds(base, RPM)],
                    out.at[c].at[pl.ds(base, RPM)])


def _norm_col(deg_row):
    deg = jnp.reshape(deg_row, (ROWB, 1))
    return jnp.where(deg > 0, lax.rsqrt(jnp.maximum(deg, 1.0)), 0.0)


def _prescale_body(feat_ref, dsrc_ref, o_ref):
    o_ref[...] = feat_ref[...] * _norm_col(dsrc_ref[...])


def _post1_body(aggs_ref, dsrc_ref, ddst_ref, w_ref, b_ref, o_ref):
    a = aggs_ref[0] + aggs_ref[1]
    z = jnp.dot(a * _norm_col(ddst_ref[...]), w_ref[...],
                preferred_element_type=jnp.float32)
    z = z + b_ref[...]
    o_ref[...] = jax.nn.silu(z) * _norm_col(dsrc_ref[...])


def _post2_body(aggs_ref, ddst_ref, w_ref, b_ref, o_ref):
    a = aggs_ref[0] + aggs_ref[1]
    z = jnp.dot(a * _norm_col(ddst_ref[...]), w_ref[...],
                preferred_element_type=jnp.float32)
    o_ref[...] = z + b_ref[...]


_deg_spec = pl.BlockSpec((1, 1, ROWB), lambda i: (i, 0, 0))
_aggs_spec = pl.BlockSpec((NC, ROWB, D), lambda i: (0, i, 0))
_rows_spec = pl.BlockSpec((ROWB, D), lambda i: (i, 0))
_w_spec = pl.BlockSpec((D, D), lambda i: (0, 0))
_b_spec = pl.BlockSpec((1, D), lambda i: (0, 0))
_out_struct = jax.ShapeDtypeStruct((N_NODES, D), jnp.float32)

_prescale = pl.pallas_call(
    _prescale_body, grid=(NBLK,),
    in_specs=[_rows_spec, _deg_spec],
    out_specs=_rows_spec, out_shape=_out_struct)

_post1 = pl.pallas_call(
    _post1_body, grid=(NBLK,),
    in_specs=[_aggs_spec, _deg_spec, _deg_spec, _w_spec, _b_spec],
    out_specs=_rows_spec, out_shape=_out_struct)

_post2 = pl.pallas_call(
    _post2_body, grid=(NBLK,),
    in_specs=[_aggs_spec, _deg_spec, _w_spec, _b_spec],
    out_specs=_rows_spec, out_shape=_out_struct)


def kernel(feat, edge_index, W1, b1, W2, b2):
    src = edge_index[0].astype(jnp.int32)
    dst = edge_index[1].astype(jnp.int32)

    pad_d = jnp.full((E_PAD - N_EDGES,), SINK, dtype=jnp.int32)
    srcd = jnp.concatenate([src, pad_d]).reshape(NT, NCHD, G)
    dstd = jnp.concatenate([dst, pad_d]).reshape(NT, NCHD, G)

    packed = src | (dst << 14)
    packed_ext = jnp.concatenate(
        [packed, jnp.full((1,), SINK << 14, jnp.int32)])
    kept = packed_ext[jnp.asarray(_POS_PAD)]
    kept0 = kept[: NS * NCH0 * G].reshape(NS, NCH0, G)
    kept1 = kept[NS * NCH0 * G:].reshape(NS, NCH1, G)

    degs = _deg_kernel(srcd, dstd).reshape(NC, 2, NPAD)
    deg = degs[0] + degs[1]
    dsrc = deg[0].reshape(NBLK, 1, ROWB)
    ddst = deg[1].reshape(NBLK, 1, ROWB)

    h1 = _prescale(feat, dsrc)
    agg1 = _mp_kernel(h1, kept0, kept1)
    h2 = _post1(agg1, dsrc, ddst, W1, b1.reshape(1, D))
    agg2 = _mp_kernel(h2, kept0, kept1)
    return _post2(agg2, ddst, W2, b2.reshape(1, D))


# speedup vs baseline: 1.0023x; 1.0023x over previous
import functools

import jax
import jax.numpy as jnp
import numpy as np
from jax import lax
from jax.experimental import pallas as pl
from jax.experimental.pallas import tpu as pltpu
from jax.experimental.pallas import tpu_sc as plsc

N_NODES = 10000
N_EDGES = 320000
D = 128
P_DROP = 0.5

NC = 2
NS = 16
NT = NC * NS
G = 128

SINK = N_NODES
NPAD = 10240
RPT = NPAD // NS
NPM = 10112
RPM = NPM // NS
ROWB = 1024
NBLK = NPAD // ROWB


def _rotl32(x, d):
    return ((x << np.uint32(d)) | (x >> np.uint32(32 - d))).astype(np.uint32)


def _threefry2x32(k0, k1, x0, x1):
    ks = [np.uint32(k0), np.uint32(k1),
          np.uint32(np.uint32(k0) ^ np.uint32(k1) ^ np.uint32(0x1BD11BDA))]
    rotations = [(13, 15, 26, 6), (17, 29, 16, 24)]
    x0 = (x0 + ks[0]).astype(np.uint32)
    x1 = (x1 + ks[1]).astype(np.uint32)
    for i in range(5):
        for r in rotations[i % 2]:
            x0 = (x0 + x1).astype(np.uint32)
            x1 = _rotl32(x1, r)
            x1 = (x1 ^ x0).astype(np.uint32)
        x0 = (x0 + ks[(i + 1) % 3]).astype(np.uint32)
        x1 = (x1 + ks[(i + 2) % 3] + np.uint32(i + 1)).astype(np.uint32)
    return x0, x1


def _drop_edge_mask(seed, n, p_keep):
    b1, b2 = _threefry2x32(np.uint32(0), np.uint32(seed),
                           np.zeros(n, dtype=np.uint32),
                           np.arange(n, dtype=np.uint32))
    bits = (b1 ^ b2).astype(np.uint32)
    u = ((bits >> np.uint32(9)) | np.uint32(0x3F800000)).view(np.float32)
    return (u - np.float32(1.0)) < np.float32(p_keep)


_MASK = _drop_edge_mask(42, N_EDGES, 1.0 - P_DROP)
_POS = np.nonzero(_MASK)[0].astype(np.int32)
K = _POS.shape[0]
NCH = -(-K // (NT * G))
K_PAD = NT * G * NCH
_POS_PAD = np.concatenate(
    [_POS, np.full(K_PAD - K, N_EDGES, dtype=np.int32)])

NCH0 = 76
NCH1 = 2 * NCH - NCH0

NCHD = -(-N_EDGES // (NT * G))
E_PAD = NT * G * NCHD


_sc_mesh = plsc.VectorSubcoreMesh(
    core_axis_name="c", subcore_axis_name="s", num_cores=NC, num_subcores=NS)


@functools.partial(
    pl.kernel,
    out_type=jax.ShapeDtypeStruct((NC * 2 * NPAD,), jnp.float32),
    mesh=_sc_mesh,
    scratch_types=[
        pltpu.VMEM((NCHD, G), jnp.int32),
        pltpu.VMEM((NCHD, G), jnp.int32),
        pltpu.VMEM((G,), jnp.float32),
        pltpu.VMEM((RPT,), jnp.float32),
        pltpu.VMEM_SHARED((NPAD,), jnp.float32),
        pltpu.VMEM_SHARED((NPAD,), jnp.float32),
    ],
)
def _deg_kernel(srcd, dstd, out, sidx, didx, ones1, zb, hs, hd):
    c = lax.axis_index("c")
    s = lax.axis_index("s")
    w = c * NS + s

    pltpu.sync_copy(srcd.at[w], sidx)
    pltpu.sync_copy(dstd.at[w], didx)

    def _fill_ones(i, _):
        ones1[pl.ds(i * 16, 16)] = jnp.ones((16,), jnp.float32)
        return 0

    def _fill_zeros(i, _):
        zb[pl.ds(i * 16, 16)] = jnp.zeros((16,), jnp.float32)
        return 0

    lax.fori_loop(0, G // 16, _fill_ones, 0)
    lax.fori_loop(0, RPT // 16, _fill_zeros, 0)

    base = s * RPT
    pltpu.sync_copy(zb, hs.at[pl.ds(base, RPT)])
    pltpu.sync_copy(zb, hd.at[pl.ds(base, RPT)])
    plsc.subcore_barrier()

    def _chunk(j, _):
        pltpu.sync_copy(ones1, hs.at[sidx.at[j]], add=True)
        pltpu.sync_copy(ones1, hd.at[didx.at[j]], add=True)
        return 0

    lax.fori_loop(0, NCHD, _chunk, 0)
    plsc.subcore_barrier()

    pltpu.sync_copy(hs.at[pl.ds(base, RPT)],
                    out.at[pl.ds(c * 2 * NPAD + base, RPT)])
    pltpu.sync_copy(hd.at[pl.ds(base, RPT)],
                    out.at[pl.ds(c * 2 * NPAD + NPAD + base, RPT)])


@functools.partial(
    pl.kernel,
    out_type=jax.ShapeDtypeStruct((NC, NPM, D), jnp.float32),
    mesh=_sc_mesh,
    scratch_types=[
        pltpu.VMEM((NCH0, G), jnp.int32),
        pltpu.VMEM((2, G), jnp.int32),
        pltpu.VMEM((2, G), jnp.int32),
        pltpu.VMEM((G, D), jnp.float32),
        pltpu.VMEM((G, D), jnp.float32),
        pltpu.VMEM_SHARED((NPM, D), jnp.float32),
        pltpu.SemaphoreType.DMA,
        pltpu.SemaphoreType.DMA,
    ],
)
def _mp_kernel(h, kept0, kept1, out,
               pval, sbuf, dbuf, rows0, rows1, agg, gsem, isem):
    c = lax.axis_index("c")
    s = lax.axis_index("s")
    ncx = jnp.where(c == 0, NCH0, NCH1)

    @pl.when(c == 0)
    def _():
        pltpu.async_copy(kept0.at[s], pval.at[pl.ds(0, NCH0)], isem)

    @pl.when(c == 1)
    def _():
        pltpu.async_copy(kept1.at[s], pval.at[pl.ds(0, NCH1)], isem)

    def _zero_row(i, _):
        for l in range(D // 16):
            rows0[i, pl.ds(l * 16, 16)] = jnp.zeros((16,), jnp.float32)
        return 0

    lax.fori_loop(0, G, _zero_row, 0)

    base = s * RPM
    for k in range(RPM // G):
        pltpu.sync_copy(rows0, agg.at[pl.ds(base + k * G, G)])
    rem = RPM % G
    if rem:
        off = base + (RPM // G) * G
        pltpu.sync_copy(rows0.at[pl.ds(0, rem)], agg.at[pl.ds(off, rem)])

    @pl.when(c == 0)
    def _():
        pltpu.make_async_copy(kept0.at[s], pval.at[pl.ds(0, NCH0)], isem).wait()

    @pl.when(c == 1)
    def _():
        pltpu.make_async_copy(kept1.at[s], pval.at[pl.ds(0, NCH1)], isem).wait()

    plsc.subcore_barrier()

    def _decode(j, b):
        for l in range(G // 16):
            v = pval[j, pl.ds(l * 16, 16)]
            sbuf[b, pl.ds(l * 16, 16)] = v & jnp.int32(0x3FFF)
            dbuf[b, pl.ds(l * 16, 16)] = lax.shift_right_logical(
                v, jnp.int32(14))

    _decode(0, 0)
    pltpu.async_copy(h.at[sbuf.at[0]], rows0, gsem)
    _decode(1, 1)
    npair = ncx // 2

    def _pair(jj, _):
        j0 = 2 * jj
        j1 = j0 + 1
        pltpu.async_copy(h.at[sbuf.at[1]], rows1, gsem)
        pltpu.make_async_copy(h.at[sbuf.at[0]], rows0, gsem).wait()
        pltpu.sync_copy(rows0, agg.at[dbuf.at[0]], add=True)

        @pl.when(jj < npair - 1)
        def _():
            _decode(j0 + 2, 0)
            pltpu.async_copy(h.at[sbuf.at[0]], rows0, gsem)

        pltpu.make_async_copy(h.at[sbuf.at[1]], rows1, gsem).wait()
        pltpu.sync_copy(rows1, agg.at[dbuf.at[1]], add=True)

        @pl.when(jj < npair - 1)
        def _():
            _decode(j1 + 2, 1)
        return 0

    lax.fori_loop(0, npair, _pair, 0)
    plsc.subcore_barrier()

    pltpu.sync_copy(agg.at[pl.ds(base, RPM)],
                    out.at[c].at[pl.ds(base, RPM)])


def _norm_col(deg_row):
    deg = jnp.reshape(deg_row, (ROWB, 1))
    return jnp.where(deg > 0, lax.rsqrt(jnp.maximum(deg, 1.0)), 0.0)


def _prescale_body(feat_ref, dsrc_ref, o_ref):
    o_ref[...] = feat_ref[...] * _norm_col(dsrc_ref[...])


def _post1_body(aggs_ref, dsrc_ref, ddst_ref, w_ref, b_ref, o_ref):
    a = aggs_ref[0] + aggs_ref[1]
    z = jnp.dot(a * _norm_col(ddst_ref[...]), w_ref[...],
                preferred_element_type=jnp.float32)
    z = z + b_ref[...]
    o_ref[...] = jax.nn.silu(z) * _norm_col(dsrc_ref[...])


def _post2_body(aggs_ref, ddst_ref, w_ref, b_ref, o_ref):
    a = aggs_ref[0] + aggs_ref[1]
    z = jnp.dot(a * _norm_col(ddst_ref[...]), w_ref[...],
                preferred_element_type=jnp.float32)
    o_ref[...] = z + b_ref[...]


_deg_spec = pl.BlockSpec((1, 1, ROWB), lambda i: (i, 0, 0))
_aggs_spec = pl.BlockSpec((NC, ROWB, D), lambda i: (0, i, 0))
_rows_spec = pl.BlockSpec((ROWB, D), lambda i: (i, 0))
_w_spec = pl.BlockSpec((D, D), lambda i: (0, 0))
_b_spec = pl.BlockSpec((1, D), lambda i: (0, 0))
_out_struct = jax.ShapeDtypeStruct((N_NODES, D), jnp.float32)

_prescale = pl.pallas_call(
    _prescale_body, grid=(NBLK,),
    in_specs=[_rows_spec, _deg_spec],
    out_specs=_rows_spec, out_shape=_out_struct)

_post1 = pl.pallas_call(
    _post1_body, grid=(NBLK,),
    in_specs=[_aggs_spec, _deg_spec, _deg_spec, _w_spec, _b_spec],
    out_specs=_rows_spec, out_shape=_out_struct)

_post2 = pl.pallas_call(
    _post2_body, grid=(NBLK,),
    in_specs=[_aggs_spec, _deg_spec, _w_spec, _b_spec],
    out_specs=_rows_spec, out_shape=_out_struct)


def kernel(feat, edge_index, W1, b1, W2, b2):
    src = edge_index[0].astype(jnp.int32)
    dst = edge_index[1].astype(jnp.int32)

    pad_d = jnp.full((E_PAD - N_EDGES,), SINK, dtype=jnp.int32)
    srcd = jnp.concatenate([src, pad_d]).reshape(NT, NCHD, G)
    dstd = jnp.concatenate([dst, pad_d]).reshape(NT, NCHD, G)

    packed = src | (dst << 14)
    packed_ext = jnp.concatenate(
        [packed, jnp.full((1,), SINK << 14, jnp.int32)])
    kept = packed_ext[jnp.asarray(_POS_PAD)]
    kept0 = kept[: NS * NCH0 * G].reshape(NS, NCH0, G)
    kept1 = kept[NS * NCH0 * G:].reshape(NS, NCH1, G)

    degs = _deg_kernel(srcd, dstd).reshape(NC, 2, NPAD)
    deg = degs[0] + degs[1]
    dsrc = deg[0].reshape(NBLK, 1, ROWB)
    ddst = deg[1].reshape(NBLK, 1, ROWB)

    h1 = _prescale(feat, dsrc)
    agg1 = _mp_kernel(h1, kept0, kept1)
    h2 = _post1(agg1, dsrc, ddst, W1, b1.reshape(1, D))
    agg2 = _mp_kernel(h2, kept0, kept1)
    return _post2(agg2, ddst, W2, b2.reshape(1, D))
